# baseline (speedup 1.0000x reference)
import functools

import jax
import jax.numpy as jnp
from jax import lax
from jax.experimental import pallas as pl
from jax.experimental.pallas import tpu as pltpu
from jax.experimental.pallas import tpu_sc as plsc

N = 10000
D = 128
E = 320000
LANES = 16
DE = D + LANES
NC, NS = 2, 16
HALF = 5120
NPAD = NC * HALF
RPT = HALF // NS
SCAN = E // NS
SSTG = 2000
NSTG = SCAN // SSTG
LIST = 20480
C = 128
NB = 2
PK = 1 << 14
BR = 1024


def _sc_agg(featx, col1, row1, ew1):
    mesh = plsc.VectorSubcoreMesh(core_axis_name="c", subcore_axis_name="s")

    @functools.partial(
        pl.kernel,
        out_type=jax.ShapeDtypeStruct((NPAD, DE), jnp.float32),
        mesh=mesh,
        compiler_params=pltpu.CompilerParams(use_tc_tiling_on_sc=False,
                                             needs_layout_passes=False),
        scratch_types=[
            pltpu.VMEM((SSTG,), jnp.int32),
            pltpu.VMEM((SSTG,), jnp.int32),
            pltpu.VMEM((SSTG,), jnp.float32),
            pltpu.VMEM((LIST,), jnp.int32),
            pltpu.VMEM((LIST,), jnp.float32),
            [pltpu.VMEM((C, DE), jnp.float32) for _ in range(NB)],
            [pltpu.VMEM((C,), jnp.int32) for _ in range(NB)],
            [pltpu.VMEM((C,), jnp.int32) for _ in range(NB)],
            pltpu.VMEM_SHARED((HALF, DE), jnp.float32),
            [pltpu.SemaphoreType.DMA for _ in range(NB)],
        ],
    )
    def body(featx_hbm, col_hbm, row_hbm, ew_hbm, out_hbm,
             scol, srow, sew, cpk, cew, gbufs, colbs, rowbs, acc, semg):
        cid = lax.axis_index("c")
        sid = lax.axis_index("s")

        lo = cid * HALF
        lo_v = jnp.full((LANES,), lo, jnp.int32)
        hi_v = lo_v + HALF

        zero16 = jnp.zeros((LANES,), jnp.float32)
        zero16i = jnp.zeros((LANES,), jnp.int32)

        def zrow(r, carry):
            for j in range(DE // LANES):
                gbufs[0][r, pl.ds(j * LANES, LANES)] = zero16
            return carry

        lax.fori_loop(0, C, zrow, 0)

        for t in range(RPT // 80):
            pltpu.sync_copy(gbufs[0].at[pl.ds(0, 80)],
                            acc.at[pl.ds(sid * RPT + t * 80, 80)])

        plsc.subcore_barrier()

        def scan_stage(j, n0):
            base = sid * SCAN + j * SSTG
            pltpu.sync_copy(col_hbm.at[pl.ds(base, SSTG)], scol)
            pltpu.sync_copy(row_hbm.at[pl.ds(base, SSTG)], srow)
            pltpu.sync_copy(ew_hbm.at[pl.ds(base, SSTG)], sew)

            def scan_group(g, n):
                col16 = scol[pl.ds(g * LANES, LANES)]
                row16 = srow[pl.ds(g * LANES, LANES)]
                ew16 = sew[pl.ds(g * LANES, LANES)]
                m = (row16 >= lo_v) & (row16 < hi_v)
                mi = lax.select(m, jnp.ones((LANES,), jnp.int32),
                                jnp.zeros((LANES,), jnp.int32))
                pc = plsc.cumsum(mi)
                pos = pc + lax.broadcast(n - 1, (LANES,))
                pk = (row16 - lo_v) * PK + col16
                plsc.store_scatter(cpk, [pos], pk, mask=m)
                plsc.store_scatter(cew, [pos], ew16, mask=m)
                return n + pc[LANES - 1]

            return lax.fori_loop(0, SSTG // LANES, scan_group, n0)

        n = lax.fori_loop(0, NSTG, scan_stage, jnp.int32(0))

        for t in range(NB * C // LANES):
            sl = pl.ds(n + t * LANES, LANES)
            cpk[sl] = zero16i
            cew[sl] = zero16

        nch = (n + C - 1) // C
        npair = jnp.maximum((nch + NB - 1) // NB, 1)
        last = npair * NB - 1

        def unpack(k, b):
            def up(g, carry):
                sl = pl.ds(g * LANES, LANES)
                v = cpk[pl.ds(k * C + g * LANES, LANES)]
                rw = v // PK
                rowbs[b][sl] = rw
                colbs[b][sl] = v - rw * PK
                return carry

            lax.fori_loop(0, C // LANES, up, 0)

        def issue_gather(b):
            pltpu.async_copy(featx_hbm.at[colbs[b]], gbufs[b], semg[b])

        def wait_gather(b):
            pltpu.make_async_copy(featx_hbm.at[colbs[b]], gbufs[b],
                                  semg[b]).wait()

        def scale(k, b):
            def group_body(g, carry):
                wvec = cew[pl.ds(k * C + g * LANES, LANES)]
                for i in range(LANES):
                    e = g * LANES + i
                    wv = lax.broadcast(wvec[i], (LANES,))
                    for j in range(DE // LANES):
                        sl = pl.ds(j * LANES, LANES)
                        gbufs[b][e, sl] = gbufs[b][e, sl] * wv
                return carry

            lax.fori_loop(0, C // LANES, group_body, 0)

        for b in range(NB):
            unpack(jnp.int32(b), b)
            issue_gather(b)

        def pair(i, carry):
            for b in range(NB):
                k = i * NB + b
                wait_gather(b)
                scale(k, b)
                pltpu.sync_copy(gbufs[b], acc.at[rowbs[b]], add=True)
                unpack(jnp.minimum(k + NB, last), b)
                issue_gather(b)
            return carry

        lax.fori_loop(0, npair, pair, 0)

        for b in range(NB):
            wait_gather(b)

        plsc.subcore_barrier()

        r0 = sid * RPT
        pltpu.sync_copy(acc.at[pl.ds(r0, RPT)],
                        out_hbm.at[pl.ds(lo + r0, RPT)])

    return body(featx, col1, row1, ew1)


def _tc_body(pa_ref, f_ref, w_ref, b_ref, o_ref):
    a = pa_ref[...]
    x = a[:, :D]
    deg = a[:, D:D + 1]
    h = x / deg
    y = lax.dot_general(h, w_ref[...], (((1,), (1,)), ((), ())),
                        preferred_element_type=jnp.float32)
    o_ref[...] = f_ref[...] + jnp.maximum(y + b_ref[...], 0.0)


def _tc_post(part, feats, W, b2):
    return pl.pallas_call(
        _tc_body,
        grid=(NPAD // BR,),
        in_specs=[
            pl.BlockSpec((BR, DE), lambda i: (i, 0)),
            pl.BlockSpec((BR, D), lambda i: (i, 0)),
            pl.BlockSpec((D, D), lambda i: (0, 0)),
            pl.BlockSpec((1, D), lambda i: (0, 0)),
        ],
        out_specs=pl.BlockSpec((BR, D), lambda i: (i, 0)),
        out_shape=jax.ShapeDtypeStruct((N, D), jnp.float32),
    )(part, feats, W, b2)


@jax.jit
def kernel(feats, edge_index, edge_weight, W, b):
    featx = jnp.concatenate(
        [feats, jnp.ones((N, LANES), jnp.float32)], axis=1)
    part = _sc_agg(featx, edge_index[1], edge_index[0], edge_weight)
    return _tc_post(part, feats, W, b.reshape(1, D))


# speedup vs baseline: 1.0003x; 1.0003x over previous
import functools

import jax
import jax.numpy as jnp
from jax import lax
from jax.experimental import pallas as pl
from jax.experimental.pallas import tpu as pltpu
from jax.experimental.pallas import tpu_sc as plsc

N = 10000
D = 128
E = 320000
LANES = 16
DE = D + LANES
NC, NS = 2, 16
HALF = 5120
NPAD = NC * HALF
RPT = HALF // NS
SCAN = E // NS
SSTG = 2000
NSTG = SCAN // SSTG
LIST = 20480
C = 128
NB = 2
PK = 1 << 14
BR = 1024


def _sc_agg(featx, col1, row1, ew1):
    mesh = plsc.VectorSubcoreMesh(core_axis_name="c", subcore_axis_name="s")

    @functools.partial(
        pl.kernel,
        out_type=jax.ShapeDtypeStruct((NPAD, DE), jnp.float32),
        mesh=mesh,
        compiler_params=pltpu.CompilerParams(use_tc_tiling_on_sc=False,
                                             needs_layout_passes=False),
        scratch_types=[
            pltpu.VMEM((SSTG,), jnp.int32),
            pltpu.VMEM((SSTG,), jnp.int32),
            pltpu.VMEM((SSTG,), jnp.float32),
            pltpu.VMEM((LIST,), jnp.int32),
            pltpu.VMEM((LIST,), jnp.float32),
            [pltpu.VMEM((C, DE), jnp.float32) for _ in range(NB)],
            [pltpu.VMEM((C,), jnp.int32) for _ in range(NB)],
            [pltpu.VMEM((C,), jnp.int32) for _ in range(NB)],
            pltpu.VMEM_SHARED((HALF, DE), jnp.float32),
            [pltpu.SemaphoreType.DMA for _ in range(NB)],
        ],
    )
    def body(featx_hbm, col_hbm, row_hbm, ew_hbm, out_hbm,
             scol, srow, sew, cpk, cew, gbufs, colbs, rowbs, acc, semg):
        cid = lax.axis_index("c")
        sid = lax.axis_index("s")

        lo = cid * HALF
        lo_v = jnp.full((LANES,), lo, jnp.int32)
        hi_v = lo_v + HALF

        zero16 = jnp.zeros((LANES,), jnp.float32)
        zero16i = jnp.zeros((LANES,), jnp.int32)

        def zrow(r, carry):
            for j in range(DE // LANES):
                gbufs[0][r, pl.ds(j * LANES, LANES)] = zero16
            return carry

        lax.fori_loop(0, C, zrow, 0)

        for t in range(RPT // 80):
            pltpu.sync_copy(gbufs[0].at[pl.ds(0, 80)],
                            acc.at[pl.ds(sid * RPT + t * 80, 80)])

        plsc.subcore_barrier()

        def scan_stage(j, n0):
            base = sid * SCAN + j * SSTG
            pltpu.sync_copy(col_hbm.at[pl.ds(base, SSTG)], scol)
            pltpu.sync_copy(row_hbm.at[pl.ds(base, SSTG)], srow)
            pltpu.sync_copy(ew_hbm.at[pl.ds(base, SSTG)], sew)

            def scan_group(g, n):
                col16 = scol[pl.ds(g * LANES, LANES)]
                row16 = srow[pl.ds(g * LANES, LANES)]
                ew16 = sew[pl.ds(g * LANES, LANES)]
                m = (row16 >= lo_v) & (row16 < hi_v)
                mi = lax.select(m, jnp.ones((LANES,), jnp.int32),
                                jnp.zeros((LANES,), jnp.int32))
                pc = plsc.cumsum(mi)
                pos = pc + lax.broadcast(n - 1, (LANES,))
                pk = (row16 - lo_v) * PK + col16
                plsc.store_scatter(cpk, [pos], pk, mask=m)
                plsc.store_scatter(cew, [pos], ew16, mask=m)
                return n + pc[LANES - 1]

            return lax.fori_loop(0, SSTG // LANES, scan_group, n0)

        n = lax.fori_loop(0, NSTG, scan_stage, jnp.int32(0))

        for t in range(NB * C // LANES):
            sl = pl.ds(n + t * LANES, LANES)
            cpk[sl] = zero16i
            cew[sl] = zero16

        nch = (n + C - 1) // C
        npair = jnp.maximum((nch + NB - 1) // NB, 1)
        last = npair * NB - 1

        def unpack(k, b):
            def up(g, carry):
                sl = pl.ds(g * LANES, LANES)
                v = cpk[pl.ds(k * C + g * LANES, LANES)]
                rowbs[b][sl] = lax.shift_right_logical(
                    v, jnp.full((LANES,), 14, jnp.int32))
                colbs[b][sl] = lax.bitwise_and(
                    v, jnp.full((LANES,), PK - 1, jnp.int32))
                return carry

            lax.fori_loop(0, C // LANES, up, 0)

        def issue_gather(b):
            pltpu.async_copy(featx_hbm.at[colbs[b]], gbufs[b], semg[b])

        def wait_gather(b):
            pltpu.make_async_copy(featx_hbm.at[colbs[b]], gbufs[b],
                                  semg[b]).wait()

        def scale(k, b):
            def group_body(g, carry):
                wvec = cew[pl.ds(k * C + g * LANES, LANES)]
                for i in range(LANES):
                    e = g * LANES + i
                    wv = lax.broadcast(wvec[i], (LANES,))
                    for j in range(DE // LANES):
                        sl = pl.ds(j * LANES, LANES)
                        gbufs[b][e, sl] = gbufs[b][e, sl] * wv
                return carry

            lax.fori_loop(0, C // LANES, group_body, 0)

        for b in range(NB):
            unpack(jnp.int32(b), b)
            issue_gather(b)

        def pair(i, carry):
            for b in range(NB):
                k = i * NB + b
                wait_gather(b)
                scale(k, b)
                pltpu.sync_copy(gbufs[b], acc.at[rowbs[b]], add=True)
                unpack(jnp.minimum(k + NB, last), b)
                issue_gather(b)
            return carry

        lax.fori_loop(0, npair, pair, 0)

        for b in range(NB):
            wait_gather(b)

        plsc.subcore_barrier()

        r0 = sid * RPT
        pltpu.sync_copy(acc.at[pl.ds(r0, RPT)],
                        out_hbm.at[pl.ds(lo + r0, RPT)])

    return body(featx, col1, row1, ew1)


def _tc_body(pa_ref, f_ref, w_ref, b_ref, o_ref):
    a = pa_ref[...]
    x = a[:, :D]
    deg = a[:, D:D + 1]
    h = x / deg
    y = lax.dot_general(h, w_ref[...], (((1,), (1,)), ((), ())),
                        preferred_element_type=jnp.float32)
    o_ref[...] = f_ref[...] + jnp.maximum(y + b_ref[...], 0.0)


def _tc_post(part, feats, W, b2):
    return pl.pallas_call(
        _tc_body,
        grid=(NPAD // BR,),
        in_specs=[
            pl.BlockSpec((BR, DE), lambda i: (i, 0)),
            pl.BlockSpec((BR, D), lambda i: (i, 0)),
            pl.BlockSpec((D, D), lambda i: (0, 0)),
            pl.BlockSpec((1, D), lambda i: (0, 0)),
        ],
        out_specs=pl.BlockSpec((BR, D), lambda i: (i, 0)),
        out_shape=jax.ShapeDtypeStruct((N, D), jnp.float32),
    )(part, feats, W, b2)


@jax.jit
def kernel(feats, edge_index, edge_weight, W, b):
    featx = jnp.concatenate(
        [feats, jnp.ones((N, LANES), jnp.float32)], axis=1)
    part = _sc_agg(featx, edge_index[1], edge_index[0], edge_weight)
    return _tc_post(part, feats, W, b.reshape(1, D))


# speedup vs baseline: 1.0579x; 1.0576x over previous
import functools

import jax
import jax.numpy as jnp
from jax import lax
from jax.experimental import pallas as pl
from jax.experimental.pallas import tpu as pltpu
from jax.experimental.pallas import tpu_sc as plsc

N = 10000
D = 128
E = 320000
LANES = 16
DE = D + LANES
NC, NS = 2, 16
HALF = 5120
NPAD = NC * HALF
RPT = HALF // NS
SCAN = E // NS
SSTG = 2000
NSTG = SCAN // SSTG
LIST = 20480
C = 128
NB = 2
PK = 1 << 14
BR = 1024


def _sc_agg(featx, col1, row1, ew1):
    mesh = plsc.VectorSubcoreMesh(core_axis_name="c", subcore_axis_name="s")

    @functools.partial(
        pl.kernel,
        out_type=(
            jax.ShapeDtypeStruct((NPAD, D), jnp.float32),
            jax.ShapeDtypeStruct((NPAD,), jnp.float32),
        ),
        mesh=mesh,
        compiler_params=pltpu.CompilerParams(needs_layout_passes=False),
        scratch_types=[
            pltpu.VMEM((SSTG,), jnp.int32),
            pltpu.VMEM((SSTG,), jnp.int32),
            pltpu.VMEM((SSTG,), jnp.float32),
            pltpu.VMEM((LIST,), jnp.int32),
            pltpu.VMEM((LIST,), jnp.float32),
            [pltpu.VMEM((C, D), jnp.float32) for _ in range(NB)],
            [pltpu.VMEM((C,), jnp.int32) for _ in range(NB)],
            [pltpu.VMEM((C,), jnp.int32) for _ in range(NB)],
            pltpu.VMEM((RPT,), jnp.float32),
            pltpu.VMEM_SHARED((HALF, D), jnp.float32),
            pltpu.VMEM_SHARED((HALF,), jnp.float32),
            [pltpu.SemaphoreType.DMA for _ in range(NB)],
        ],
    )
    def body(featx_hbm, col_hbm, row_hbm, ew_hbm, out_hbm, deg_hbm,
             scol, srow, sew, cpk, cew, gbufs, colbs, rowbs, zdbuf,
             acc, dacc, semg):
        cid = lax.axis_index("c")
        sid = lax.axis_index("s")

        lo = cid * HALF
        lo_v = jnp.full((LANES,), lo, jnp.int32)
        hi_v = lo_v + HALF

        zero16 = jnp.zeros((LANES,), jnp.float32)
        zero16i = jnp.zeros((LANES,), jnp.int32)

        def zrow(r, carry):
            for j in range(D // LANES):
                gbufs[0][r, pl.ds(j * LANES, LANES)] = zero16
            return carry

        lax.fori_loop(0, C, zrow, 0)

        for t in range(RPT // 80):
            pltpu.sync_copy(gbufs[0].at[pl.ds(0, 80)],
                            acc.at[pl.ds(sid * RPT + t * 80, 80)])

        def zdeg(r, carry):
            zdbuf[pl.ds(r * LANES, LANES)] = zero16
            return carry

        lax.fori_loop(0, RPT // LANES, zdeg, 0)
        pltpu.sync_copy(zdbuf, dacc.at[pl.ds(sid * RPT, RPT)])

        plsc.subcore_barrier()

        def scan_stage(j, n0):
            base = sid * SCAN + j * SSTG
            pltpu.sync_copy(col_hbm.at[pl.ds(base, SSTG)], scol)
            pltpu.sync_copy(row_hbm.at[pl.ds(base, SSTG)], srow)
            pltpu.sync_copy(ew_hbm.at[pl.ds(base, SSTG)], sew)

            def scan_group(g, n):
                col16 = scol[pl.ds(g * LANES, LANES)]
                row16 = srow[pl.ds(g * LANES, LANES)]
                ew16 = sew[pl.ds(g * LANES, LANES)]
                m = (row16 >= lo_v) & (row16 < hi_v)
                mi = lax.select(m, jnp.ones((LANES,), jnp.int32),
                                jnp.zeros((LANES,), jnp.int32))
                pc = plsc.cumsum(mi)
                pos = pc + lax.broadcast(n - 1, (LANES,))
                pk = (row16 - lo_v) * PK + col16
                plsc.store_scatter(cpk, [pos], pk, mask=m)
                plsc.store_scatter(cew, [pos], ew16, mask=m)
                return n + pc[LANES - 1]

            return lax.fori_loop(0, SSTG // LANES, scan_group, n0)

        n = lax.fori_loop(0, NSTG, scan_stage, jnp.int32(0))

        for t in range(NB * C // LANES):
            sl = pl.ds(n + t * LANES, LANES)
            cpk[sl] = zero16i
            cew[sl] = zero16

        nch = (n + C - 1) // C
        npair = jnp.maximum((nch + NB - 1) // NB, 1)
        last = npair * NB - 1

        def unpack(k, b):
            def up(g, carry):
                sl = pl.ds(g * LANES, LANES)
                v = cpk[pl.ds(k * C + g * LANES, LANES)]
                rowbs[b][sl] = lax.shift_right_logical(
                    v, jnp.full((LANES,), 14, jnp.int32))
                colbs[b][sl] = lax.bitwise_and(
                    v, jnp.full((LANES,), PK - 1, jnp.int32))
                return carry

            lax.fori_loop(0, C // LANES, up, 0)

        def issue_gather(b):
            pltpu.async_copy(featx_hbm.at[colbs[b]], gbufs[b], semg[b])

        def wait_gather(b):
            pltpu.make_async_copy(featx_hbm.at[colbs[b]], gbufs[b],
                                  semg[b]).wait()

        def scale(k, b):
            def group_body(g, carry):
                wvec = cew[pl.ds(k * C + g * LANES, LANES)]
                for i in range(LANES):
                    e = g * LANES + i
                    wv = lax.broadcast(wvec[i], (LANES,))
                    for j in range(D // LANES):
                        sl = pl.ds(j * LANES, LANES)
                        gbufs[b][e, sl] = gbufs[b][e, sl] * wv
                return carry

            lax.fori_loop(0, C // LANES, group_body, 0)

        for b in range(NB):
            unpack(jnp.int32(b), b)
            issue_gather(b)

        def pair(i, carry):
            for b in range(NB):
                k = i * NB + b
                wait_gather(b)
                scale(k, b)
                pltpu.sync_copy(gbufs[b], acc.at[rowbs[b]], add=True)
                pltpu.sync_copy(cew.at[pl.ds(k * C, C)],
                                dacc.at[rowbs[b]], add=True)
                unpack(jnp.minimum(k + NB, last), b)
                issue_gather(b)
            return carry

        lax.fori_loop(0, npair, pair, 0)

        for b in range(NB):
            wait_gather(b)

        plsc.subcore_barrier()

        r0 = sid * RPT
        pltpu.sync_copy(acc.at[pl.ds(r0, RPT)],
                        out_hbm.at[pl.ds(lo + r0, RPT)])
        pltpu.sync_copy(dacc.at[pl.ds(r0, RPT)], zdbuf)
        pltpu.sync_copy(zdbuf, deg_hbm.at[pl.ds(lo + r0, RPT)])

    return body(featx, col1, row1, ew1)


def _tc_body(pa_ref, dp_ref, f_ref, w_ref, b_ref, o_ref):
    x = pa_ref[...]
    deg = dp_ref[...]
    h = x / deg
    y = lax.dot_general(h, w_ref[...], (((1,), (1,)), ((), ())),
                        preferred_element_type=jnp.float32)
    o_ref[...] = f_ref[...] + jnp.maximum(y + b_ref[...], 0.0)


def _tc_post(part, degp, feats, W, b2):
    return pl.pallas_call(
        _tc_body,
        grid=(NPAD // BR,),
        in_specs=[
            pl.BlockSpec((BR, D), lambda i: (i, 0)),
            pl.BlockSpec((BR, 1), lambda i: (i, 0)),
            pl.BlockSpec((BR, D), lambda i: (i, 0)),
            pl.BlockSpec((D, D), lambda i: (0, 0)),
            pl.BlockSpec((1, D), lambda i: (0, 0)),
        ],
        out_specs=pl.BlockSpec((BR, D), lambda i: (i, 0)),
        out_shape=jax.ShapeDtypeStruct((N, D), jnp.float32),
    )(part, degp, feats, W, b2)


@jax.jit
def kernel(feats, edge_index, edge_weight, W, b):
    part, degp = _sc_agg(feats, edge_index[1], edge_index[0], edge_weight)
    return _tc_post(part, degp.reshape(NPAD, 1), feats, W, b.reshape(1, D))


# speedup vs baseline: 1.6987x; 1.6057x over previous
import functools

import jax
import jax.numpy as jnp
from jax import lax
from jax.experimental import pallas as pl
from jax.experimental.pallas import tpu as pltpu
from jax.experimental.pallas import tpu_sc as plsc

N = 10000
D = 128
E = 320000
LANES = 16
NC, NS = 2, 16
HALF = 5120
NPAD = NC * HALF
RPT = HALF // NS
SCAN = E // NS
SSTG = 2000
NSTG = SCAN // SSTG
LIST = 20480
C = 128
BR = 1024


def _sc_agg(feats, col1, row1, ew1):
    mesh = plsc.VectorSubcoreMesh(core_axis_name="c", subcore_axis_name="s")

    @functools.partial(
        pl.kernel,
        out_type=(
            jax.ShapeDtypeStruct((NPAD, D), jnp.float32),
            jax.ShapeDtypeStruct((NPAD,), jnp.float32),
        ),
        mesh=mesh,
        compiler_params=pltpu.CompilerParams(use_tc_tiling_on_sc=False,
                                             needs_layout_passes=False),
        scratch_types=[
            pltpu.VMEM((SSTG,), jnp.int32),
            pltpu.VMEM((SSTG,), jnp.int32),
            pltpu.VMEM((SSTG,), jnp.float32),
            pltpu.VMEM((LIST,), jnp.int32),
            pltpu.VMEM((LIST,), jnp.int32),
            pltpu.VMEM((LIST,), jnp.float32),
            pltpu.VMEM((C, D), jnp.float32),
            pltpu.VMEM((C,), jnp.int32),
            pltpu.VMEM((C,), jnp.float32),
            pltpu.VMEM((RPT,), jnp.float32),
            pltpu.VMEM_SHARED((HALF, D), jnp.float32),
            pltpu.VMEM_SHARED((HALF,), jnp.float32),
            pltpu.SemaphoreType.DMA,
        ],
    )
    def body(feats_hbm, col_hbm, row_hbm, ew_hbm, out_hbm, deg_hbm,
             scol, srow, sew, ccol, crow, cew, gbuf, rowb, ewb, zdbuf,
             acc, dacc, sem):
        cid = lax.axis_index("c")
        sid = lax.axis_index("s")

        lo = cid * HALF
        lo_v = jnp.full((LANES,), lo, jnp.int32)
        hi_v = lo_v + HALF

        zero16 = jnp.zeros((LANES,), jnp.float32)

        def zrow(r, carry):
            for j in range(D // LANES):
                gbuf[r, pl.ds(j * LANES, LANES)] = zero16
            return carry

        lax.fori_loop(0, C, zrow, 0)

        def zdeg(r, carry):
            zdbuf[pl.ds(r * LANES, LANES)] = zero16
            return carry

        lax.fori_loop(0, RPT // LANES, zdeg, 0)

        for t in range(RPT // 64):
            pltpu.sync_copy(gbuf.at[pl.ds(0, 64)],
                            acc.at[pl.ds(sid * RPT + t * 64, 64)])
        pltpu.sync_copy(zdbuf, dacc.at[pl.ds(sid * RPT, RPT)])

        plsc.subcore_barrier()

        def scan_stage(j, n0):
            base = sid * SCAN + j * SSTG
            pltpu.sync_copy(col_hbm.at[pl.ds(base, SSTG)], scol)
            pltpu.sync_copy(row_hbm.at[pl.ds(base, SSTG)], srow)
            pltpu.sync_copy(ew_hbm.at[pl.ds(base, SSTG)], sew)

            def scan_group(g, n):
                col16 = scol[pl.ds(g * LANES, LANES)]
                row16 = srow[pl.ds(g * LANES, LANES)]
                ew16 = sew[pl.ds(g * LANES, LANES)]
                m = (row16 >= lo_v) & (row16 < hi_v)
                mi = lax.select(m, jnp.ones((LANES,), jnp.int32),
                                jnp.zeros((LANES,), jnp.int32))
                pc = plsc.cumsum(mi)
                pos = pc + lax.broadcast(n - 1, (LANES,))
                plsc.store_scatter(ccol, [pos], col16, mask=m)
                plsc.store_scatter(crow, [pos], row16 - lo_v, mask=m)
                plsc.store_scatter(cew, [pos], ew16, mask=m)
                return n + pc[LANES - 1]

            return lax.fori_loop(0, SSTG // LANES, scan_group, n0)

        n = lax.fori_loop(0, NSTG, scan_stage, jnp.int32(0))

        zero16i = jnp.zeros((LANES,), jnp.int32)
        for t in range(C // LANES):
            sl = pl.ds(n + t * LANES, LANES)
            ccol[sl] = zero16i
            crow[sl] = zero16i
            cew[sl] = zero16

        def group_body(k, g, carry):
            base = k * C + g * LANES
            wvec = cew[pl.ds(base, LANES)]
            ewb[pl.ds(g * LANES, LANES)] = wvec
            rowb[pl.ds(g * LANES, LANES)] = crow[pl.ds(base, LANES)]
            for i in range(LANES):
                e = g * LANES + i
                wv = lax.broadcast(wvec[i], (LANES,))
                for j in range(D // LANES):
                    sl = pl.ds(j * LANES, LANES)
                    gbuf[e, sl] = gbuf[e, sl] * wv
            return carry

        def chunk_body(k, carry):
            pltpu.async_copy(feats_hbm.at[ccol.at[pl.ds(k * C, C)]],
                             gbuf, sem).wait()
            lax.fori_loop(0, C // LANES, functools.partial(group_body, k), 0)
            pltpu.sync_copy(gbuf, acc.at[rowb], add=True)
            pltpu.sync_copy(ewb, dacc.at[rowb], add=True)
            return carry

        nch = (n + C - 1) // C
        lax.fori_loop(0, nch, chunk_body, 0)

        plsc.subcore_barrier()

        r0 = sid * RPT
        pltpu.sync_copy(acc.at[pl.ds(r0, RPT)],
                        out_hbm.at[pl.ds(lo + r0, RPT)])
        pltpu.sync_copy(dacc.at[pl.ds(r0, RPT)],
                        deg_hbm.at[pl.ds(lo + r0, RPT)])

    return body(feats, col1, row1, ew1)


def _tc_body(pa_ref, dp_ref, f_ref, w_ref, b_ref, o_ref):
    x = pa_ref[...]
    deg = dp_ref[...]
    h = x / deg
    y = lax.dot_general(h, w_ref[...], (((1,), (1,)), ((), ())),
                        preferred_element_type=jnp.float32)
    o_ref[...] = f_ref[...] + jnp.maximum(y + b_ref[...], 0.0)


def _tc_post(part, degp, feats, W, b2):
    return pl.pallas_call(
        _tc_body,
        grid=(NPAD // BR,),
        in_specs=[
            pl.BlockSpec((BR, D), lambda i: (i, 0)),
            pl.BlockSpec((BR, 1), lambda i: (i, 0)),
            pl.BlockSpec((BR, D), lambda i: (i, 0)),
            pl.BlockSpec((D, D), lambda i: (0, 0)),
            pl.BlockSpec((1, D), lambda i: (0, 0)),
        ],
        out_specs=pl.BlockSpec((BR, D), lambda i: (i, 0)),
        out_shape=jax.ShapeDtypeStruct((N, D), jnp.float32),
    )(part, degp, feats, W, b2)


@jax.jit
def kernel(feats, edge_index, edge_weight, W, b):
    part, degp = _sc_agg(feats, edge_index[1], edge_index[0], edge_weight)
    return _tc_post(part, degp.reshape(NPAD, 1), feats, W, b.reshape(1, D))
